# baseline (speedup 1.0000x reference)
import dataclasses
import functools

import jax
import jax.numpy as jnp
from jax import lax
from jax.experimental import pallas as pl
from jax.experimental.pallas import tpu as pltpu
from jax.experimental.pallas import tpu_sc as plsc

_B = 8192
_NW = 32
_RPW = _B // _NW
_CH = 8
_NBATCH = _RPW // 16


def _sc_compiler_params():
    cp = pltpu.CompilerParams()
    if "needs_layout_passes" in pltpu.CompilerParams.__dataclass_fields__:
        cp = dataclasses.replace(cp, needs_layout_passes=False)
    return cp


def _sc_bandwidthify(index):
    mesh = plsc.VectorSubcoreMesh(core_axis_name="c", subcore_axis_name="s")

    @functools.partial(
        pl.kernel,
        out_type=jax.ShapeDtypeStruct((_B, _B), jnp.float32),
        mesh=mesh,
        compiler_params=_sc_compiler_params(),
        scratch_types=[
            pltpu.VMEM((_B,), jnp.float32),
            pltpu.VMEM((_CH, _B), jnp.float32),
            pltpu.SemaphoreType.DMA,
        ],
    )
    def k(idx_hbm, out_hbm, idx_v, buf, sem):
        wid = lax.axis_index("s") * 2 + lax.axis_index("c")
        base = wid * _RPW
        pltpu.sync_copy(idx_hbm, idx_v)
        zero16 = jnp.zeros((16,), jnp.float32)

        @pl.loop(0, _B // 16)
        def _(j):
            for r in range(_CH):
                buf[r, pl.ds(j * 16, 16)] = zero16

        lane = lax.iota(jnp.int32, 16)
        rl = lane & (_CH - 1)
        one_i = jnp.ones((16,), jnp.int32)
        zero_i = jnp.zeros((16,), jnp.int32)
        one_f = jnp.ones((16,), jnp.float32)
        cap = jnp.full((16,), _B - 1, jnp.int32)

        @pl.loop(0, _NBATCH)
        def _(b):
            tok0 = base + b * 16
            x = idx_v[pl.ds(tok0, 16)]
            i1 = x.astype(jnp.int32)
            fr = x - i1.astype(jnp.float32)
            i2 = jnp.minimum(i1 + jnp.where(fr > 0, one_i, zero_i), cap)
            g1 = plsc.load_gather(idx_v, [i1])
            g2 = plsc.load_gather(idx_v, [i2])
            v1 = g1 - g1.astype(jnp.int32).astype(jnp.float32)
            v2 = 1.0 - (g2 - g2.astype(jnp.int32).astype(jnp.float32))
            eq = i1 == i2
            v1 = jnp.where(eq, one_f, v1)
            v2 = jnp.where(eq, one_f, v2)
            for c in range(2):
                mask = (lane >> 3) == c
                plsc.store_scatter(buf, [rl, i1], v1, mask=mask)
                plsc.store_scatter(buf, [rl, i2], v2, mask=mask)
                dst = out_hbm.at[pl.ds(tok0 + c * _CH, _CH)]
                cp = pltpu.make_async_copy(buf, dst, sem)
                cp.start()
                cp.wait()
                plsc.store_scatter(buf, [rl, i1], zero16, mask=mask)
                plsc.store_scatter(buf, [rl, i2], zero16, mask=mask)

    return k(index)


def kernel(index):
    return _sc_bandwidthify(index)
